# baseline (speedup 1.0000x reference)
import functools

import jax
import jax.numpy as jnp
from jax import lax
from jax.experimental import pallas as pl
from jax.experimental.pallas import tpu as pltpu
from jax.experimental.pallas import tpu_sc as plsc

N = 16384
C = 1000
GAMMA = 2.0
IGNORE_ID = -1

NC = 2
NS = 16
NW = NC * NS
ROWS_PER_W = N // NW
BLK = 32
NBLK = ROWS_PER_W // BLK
CP = 1008
NFULL = C // 16
TAIL_BASE = NFULL * 16

_LN2 = 0.6931471805599453


def _ln(s):
    bits = lax.bitcast_convert_type(s, jnp.int32)
    e = ((bits >> 23) & 0xFF) - 127
    mbits = (bits & 0x007FFFFF) | 0x3F800000
    m = lax.bitcast_convert_type(mbits, jnp.float32)
    big = m > 1.4142135381698608
    m = jnp.where(big, m * 0.5, m)
    e = e + jnp.where(big, 1, 0)
    z = (m - 1.0) / (m + 1.0)
    z2 = z * z
    poly = 1.0 + z2 * (0.3333333432674408 + z2 * (0.20000000298023224 + z2 * 0.14285714924335480))
    lnm = 2.0 * z * poly
    return e.astype(jnp.float32) * _LN2 + lnm


def _body(x_hbm, t_hbm, a_hbm, out_hbm, buf, tgt, alo, ob):
    wid = lax.axis_index("s") * NC + lax.axis_index("c")
    row0 = wid * ROWS_PER_W
    iota = lax.iota(jnp.int32, 16)

    pltpu.sync_copy(t_hbm.at[pl.ds(row0, ROWS_PER_W)], tgt)
    pltpu.sync_copy(a_hbm, alo)

    def grp_body(b, g, carry):
        acc, cnt = carry

        tail_ok = iota < (C - TAIL_BASE)

        def row_body(i, mv_sv):
            maxv, sumv = mv_sv
            r = g * 16 + i

            def p1(k, mv):
                x = buf[r, pl.ds(pl.multiple_of(k * 16, 16), 16)]
                return jnp.maximum(mv, x)

            mv = lax.fori_loop(0, NFULL, p1, jnp.full((16,), -jnp.inf, jnp.float32))
            xt_ = buf[r, pl.ds(TAIL_BASE, 16)]
            mv = jnp.maximum(mv, jnp.where(tail_ok, xt_, -jnp.inf))
            rmax = jnp.max(mv)

            def p2(k, sv):
                x = buf[r, pl.ds(pl.multiple_of(k * 16, 16), 16)]
                return sv + jnp.exp(x - rmax)

            sv = lax.fori_loop(0, NFULL, p2, jnp.zeros((16,), jnp.float32))
            sv = sv + jnp.where(tail_ok, jnp.exp(xt_ - rmax), 0.0)
            sel = iota == i
            maxv = jnp.where(sel, rmax, maxv)
            sumv = jnp.where(sel, jnp.sum(sv), sumv)
            return maxv, sumv

        zero = jnp.zeros((16,), jnp.float32)
        mx, sm = lax.fori_loop(0, 16, row_body, (zero, zero))

        t = tgt[pl.ds(pl.multiple_of(b * BLK + g * 16, 16), 16)]
        valid = t != IGNORE_ID
        ts = jnp.where(valid, t, 0)
        rows = g * 16 + iota
        xt = plsc.load_gather(buf, [rows, ts])
        logp = (xt - mx) - _ln(sm)
        p = jnp.exp(logp)
        av = plsc.load_gather(alo, [ts])
        om = 1.0 - p
        loss = -av * om * om * logp
        acc = acc + jnp.where(valid, loss, 0.0)
        cnt = cnt + jnp.where(valid, 1.0, 0.0)
        return acc, cnt

    def blk_body(b, carry):
        base = row0 + b * BLK
        pltpu.sync_copy(x_hbm.at[pl.ds(base, BLK)], buf.at[:, pl.ds(0, C)])
        return lax.fori_loop(0, BLK // 16, functools.partial(grp_body, b), carry)

    zero = jnp.zeros((16,), jnp.float32)
    acc, cnt = lax.fori_loop(0, NBLK, blk_body, (zero, zero))
    ob[0, :] = acc
    ob[1, :] = cnt
    pltpu.sync_copy(ob, out_hbm.at[wid])


@jax.jit
def _focal_partials(inputs, targets, alpha_flat):
    mesh = plsc.VectorSubcoreMesh(core_axis_name="c", subcore_axis_name="s")
    f = pl.kernel(
        _body,
        out_type=jax.ShapeDtypeStruct((NW, 2, 16), jnp.float32),
        mesh=mesh,
        compiler_params=pltpu.CompilerParams(
            use_tc_tiling_on_sc=False, needs_layout_passes=False
        ),
        scratch_types=[
            pltpu.VMEM((BLK, CP), jnp.float32),
            pltpu.VMEM((ROWS_PER_W,), jnp.int32),
            pltpu.VMEM((C,), jnp.float32),
            pltpu.VMEM((2, 16), jnp.float32),
        ],
    )
    return f(inputs, targets, alpha_flat)


def kernel(inputs, targets, alpha):
    out = _focal_partials(inputs, targets, alpha.reshape(C))
    loss_sum = out[:, 0, :].sum()
    cnt = out[:, 1, :].sum()
    return loss_sum / jnp.maximum(cnt, 1.0)


# speedup vs baseline: 1.9462x; 1.9462x over previous
import functools

import jax
import jax.numpy as jnp
from jax import lax
from jax.experimental import pallas as pl
from jax.experimental.pallas import tpu as pltpu
from jax.experimental.pallas import tpu_sc as plsc

N = 16384
C = 1000
GAMMA = 2.0
IGNORE_ID = -1

NC = 2
NS = 16
NW = NC * NS
ROWS_PER_W = N // NW
BLK = 32
NBLK = ROWS_PER_W // BLK
CP = 1008
NFULL = C // 16
TAIL_BASE = NFULL * 16

_LN2 = 0.6931471805599453


def _ln(s):
    bits = lax.bitcast_convert_type(s, jnp.int32)
    e = ((bits >> 23) & 0xFF) - 127
    mbits = (bits & 0x007FFFFF) | 0x3F800000
    m = lax.bitcast_convert_type(mbits, jnp.float32)
    big = m > 1.4142135381698608
    m = jnp.where(big, m * 0.5, m)
    e = e + jnp.where(big, 1, 0)
    z = (m - 1.0) / (m + 1.0)
    z2 = z * z
    poly = 1.0 + z2 * (0.3333333432674408 + z2 * (0.20000000298023224 + z2 * 0.14285714924335480))
    lnm = 2.0 * z * poly
    return e.astype(jnp.float32) * _LN2 + lnm


def _body(x_hbm, t_hbm, a_hbm, out_hbm, buf, tgt, alo, ob):
    wid = lax.axis_index("s") * NC + lax.axis_index("c")
    row0 = wid * ROWS_PER_W
    iota = lax.iota(jnp.int32, 16)

    pltpu.sync_copy(t_hbm.at[pl.ds(row0, ROWS_PER_W)], tgt)
    pltpu.sync_copy(a_hbm, alo)

    def grp_body(b, g, carry):
        acc, cnt = carry

        tail_ok = iota < (C - TAIL_BASE)

        def row_body(i, mv_sv):
            maxv, sumv = mv_sv
            r = g * 16 + i

            NACC = 4
            maccs = [None] * NACC
            for k in range(NFULL):
                x = buf[r, pl.ds(k * 16, 16)]
                a = maccs[k % NACC]
                maccs[k % NACC] = x if a is None else jnp.maximum(a, x)
            xt_ = buf[r, pl.ds(TAIL_BASE, 16)]
            maccs[NFULL % NACC] = jnp.maximum(
                maccs[NFULL % NACC], jnp.where(tail_ok, xt_, -jnp.inf)
            )
            mv = maccs[0]
            for a in maccs[1:]:
                mv = jnp.maximum(mv, a)
            rmax = jnp.max(mv)

            saccs = [jnp.zeros((16,), jnp.float32) for _ in range(NACC)]
            for k in range(NFULL):
                x = buf[r, pl.ds(k * 16, 16)]
                saccs[k % NACC] = saccs[k % NACC] + jnp.exp(x - rmax)
            saccs[NFULL % NACC] = saccs[NFULL % NACC] + jnp.where(
                tail_ok, jnp.exp(xt_ - rmax), 0.0
            )
            sv = (saccs[0] + saccs[1]) + (saccs[2] + saccs[3])
            sel = iota == i
            maxv = jnp.where(sel, rmax, maxv)
            sumv = jnp.where(sel, jnp.sum(sv), sumv)
            return maxv, sumv

        zero = jnp.zeros((16,), jnp.float32)
        mx, sm = lax.fori_loop(0, 16, row_body, (zero, zero))

        t = tgt[pl.ds(pl.multiple_of(b * BLK + g * 16, 16), 16)]
        valid = t != IGNORE_ID
        ts = jnp.where(valid, t, 0)
        rows = g * 16 + iota
        xt = plsc.load_gather(buf, [rows, ts])
        logp = (xt - mx) - _ln(sm)
        p = jnp.exp(logp)
        av = plsc.load_gather(alo, [ts])
        om = 1.0 - p
        loss = -av * om * om * logp
        acc = acc + jnp.where(valid, loss, 0.0)
        cnt = cnt + jnp.where(valid, 1.0, 0.0)
        return acc, cnt

    def blk_body(b, carry):
        base = row0 + b * BLK
        pltpu.sync_copy(x_hbm.at[pl.ds(base, BLK)], buf.at[:, pl.ds(0, C)])
        return lax.fori_loop(0, BLK // 16, functools.partial(grp_body, b), carry)

    zero = jnp.zeros((16,), jnp.float32)
    acc, cnt = lax.fori_loop(0, NBLK, blk_body, (zero, zero))
    ob[0, :] = acc
    ob[1, :] = cnt
    pltpu.sync_copy(ob, out_hbm.at[wid])


@jax.jit
def _focal_partials(inputs, targets, alpha_flat):
    mesh = plsc.VectorSubcoreMesh(core_axis_name="c", subcore_axis_name="s")
    f = pl.kernel(
        _body,
        out_type=jax.ShapeDtypeStruct((NW, 2, 16), jnp.float32),
        mesh=mesh,
        compiler_params=pltpu.CompilerParams(
            use_tc_tiling_on_sc=False, needs_layout_passes=False
        ),
        scratch_types=[
            pltpu.VMEM((BLK, CP), jnp.float32),
            pltpu.VMEM((ROWS_PER_W,), jnp.int32),
            pltpu.VMEM((C,), jnp.float32),
            pltpu.VMEM((2, 16), jnp.float32),
        ],
    )
    return f(inputs, targets, alpha_flat)


def kernel(inputs, targets, alpha):
    out = _focal_partials(inputs, targets, alpha.reshape(C))
    loss_sum = out[:, 0, :].sum()
    cnt = out[:, 1, :].sum()
    return loss_sum / jnp.maximum(cnt, 1.0)


# speedup vs baseline: 2.1924x; 1.1265x over previous
import functools

import jax
import jax.numpy as jnp
from jax import lax
from jax.experimental import pallas as pl
from jax.experimental.pallas import tpu as pltpu
from jax.experimental.pallas import tpu_sc as plsc

N = 16384
C = 1000
GAMMA = 2.0
IGNORE_ID = -1

NC = 2
NS = 16
NW = NC * NS
ROWS_PER_W = N // NW
BLK = 32
NBLK = ROWS_PER_W // BLK
CP = 1008
NFULL = C // 16
TAIL_BASE = NFULL * 16

_LN2 = 0.6931471805599453


def _ln(s):
    bits = lax.bitcast_convert_type(s, jnp.int32)
    e = ((bits >> 23) & 0xFF) - 127
    mbits = (bits & 0x007FFFFF) | 0x3F800000
    m = lax.bitcast_convert_type(mbits, jnp.float32)
    big = m > 1.4142135381698608
    m = jnp.where(big, m * 0.5, m)
    e = e + jnp.where(big, 1, 0)
    z = (m - 1.0) / (m + 1.0)
    z2 = z * z
    poly = 1.0 + z2 * (0.3333333432674408 + z2 * (0.20000000298023224 + z2 * 0.14285714924335480))
    lnm = 2.0 * z * poly
    return e.astype(jnp.float32) * _LN2 + lnm


def _body(x_hbm, t_hbm, a_hbm, out_hbm, bufs, tgt, alo, ob, sem):
    wid = lax.axis_index("s") * NC + lax.axis_index("c")
    row0 = wid * ROWS_PER_W
    iota = lax.iota(jnp.int32, 16)

    pltpu.sync_copy(t_hbm.at[pl.ds(row0, ROWS_PER_W)], tgt)
    pltpu.sync_copy(a_hbm, alo)

    def start_blk(b, par):
        pltpu.async_copy(
            x_hbm.at[pl.ds(row0 + b * BLK, BLK)],
            bufs.at[par, :, pl.ds(0, C)],
            sem.at[par],
        )

    def wait_blk(b, par):
        pltpu.make_async_copy(
            x_hbm.at[pl.ds(row0 + b * BLK, BLK)],
            bufs.at[par, :, pl.ds(0, C)],
            sem.at[par],
        ).wait()

    def grp_body(buf, b, g, carry):
        acc, cnt = carry

        tail_ok = iota < (C - TAIL_BASE)

        def row_body(i, mv_sv):
            maxv, sumv = mv_sv
            r = g * 16 + i

            NACC = 4
            maccs = [None] * NACC
            for k in range(NFULL):
                x = buf[r, pl.ds(k * 16, 16)]
                a = maccs[k % NACC]
                maccs[k % NACC] = x if a is None else jnp.maximum(a, x)
            xt_ = buf[r, pl.ds(TAIL_BASE, 16)]
            maccs[NFULL % NACC] = jnp.maximum(
                maccs[NFULL % NACC], jnp.where(tail_ok, xt_, -jnp.inf)
            )
            mv = maccs[0]
            for a in maccs[1:]:
                mv = jnp.maximum(mv, a)
            rmax = jnp.max(mv)

            saccs = [jnp.zeros((16,), jnp.float32) for _ in range(NACC)]
            for k in range(NFULL):
                x = buf[r, pl.ds(k * 16, 16)]
                saccs[k % NACC] = saccs[k % NACC] + jnp.exp(x - rmax)
            saccs[NFULL % NACC] = saccs[NFULL % NACC] + jnp.where(
                tail_ok, jnp.exp(xt_ - rmax), 0.0
            )
            sv = (saccs[0] + saccs[1]) + (saccs[2] + saccs[3])
            sel = iota == i
            maxv = jnp.where(sel, rmax, maxv)
            sumv = jnp.where(sel, jnp.sum(sv), sumv)
            return maxv, sumv

        zero = jnp.zeros((16,), jnp.float32)
        mx, sm = lax.fori_loop(0, 16, row_body, (zero, zero))

        t = tgt[pl.ds(pl.multiple_of(b * BLK + g * 16, 16), 16)]
        valid = t != IGNORE_ID
        ts = jnp.where(valid, t, 0)
        rows = g * 16 + iota
        xt = plsc.load_gather(buf, [rows, ts])
        logp = (xt - mx) - _ln(sm)
        p = jnp.exp(logp)
        av = plsc.load_gather(alo, [ts])
        om = 1.0 - p
        loss = -av * om * om * logp
        acc = acc + jnp.where(valid, loss, 0.0)
        cnt = cnt + jnp.where(valid, 1.0, 0.0)
        return acc, cnt

    def blk_body(b, carry):
        par = lax.rem(b, 2)
        wait_blk(b, par)

        @pl.when(b + 1 < NBLK)
        def _():
            start_blk(b + 1, 1 - par)

        buf = bufs.at[par]
        return lax.fori_loop(
            0, BLK // 16, functools.partial(grp_body, buf, b), carry
        )

    start_blk(0, 0)
    zero = jnp.zeros((16,), jnp.float32)
    acc, cnt = lax.fori_loop(0, NBLK, blk_body, (zero, zero))
    ob[0, :] = acc
    ob[1, :] = cnt
    pltpu.sync_copy(ob, out_hbm.at[wid])


@jax.jit
def _focal_partials(inputs, targets, alpha_flat):
    mesh = plsc.VectorSubcoreMesh(core_axis_name="c", subcore_axis_name="s")
    f = pl.kernel(
        _body,
        out_type=jax.ShapeDtypeStruct((NW, 2, 16), jnp.float32),
        mesh=mesh,
        compiler_params=pltpu.CompilerParams(
            use_tc_tiling_on_sc=False, needs_layout_passes=False
        ),
        scratch_types=[
            pltpu.VMEM((2, BLK, CP), jnp.float32),
            pltpu.VMEM((ROWS_PER_W,), jnp.int32),
            pltpu.VMEM((C,), jnp.float32),
            pltpu.VMEM((2, 16), jnp.float32),
            pltpu.SemaphoreType.DMA((2,)),
        ],
    )
    return f(inputs, targets, alpha_flat)


def kernel(inputs, targets, alpha):
    out = _focal_partials(inputs, targets, alpha.reshape(C))
    loss_sum = out[:, 0, :].sum()
    cnt = out[:, 1, :].sum()
    return loss_sum / jnp.maximum(cnt, 1.0)
